# baseline (speedup 1.0000x reference)
import functools

import jax
import jax.numpy as jnp
from jax import lax
from jax.experimental import pallas as pl
from jax.experimental.pallas import tpu as pltpu
from jax.experimental.pallas import tpu_sc as plsc

E = 160000
G = 50
F = 128
X = 8
BM = 2048
E_PAD = ((E + X * (BM - 1)) // BM) * BM
NB = E_PAD // BM
NW = 32
CHUNK = E // NW
NT = CHUNK // 128
TAIL = CHUNK - NT * 128
ROWS_2D = E // 128
SHIFT = 0.6931471805599453


def _iscan_rows(x):
    n = x.shape[0]
    d = 1
    while d < n:
        pad = jnp.zeros((d,) + x.shape[1:], x.dtype)
        x = x + jnp.concatenate([pad, x[:-d]], axis=0)
        d *= 2
    return x


def _iscan_lanes(x):
    n = x.shape[1]
    d = 1
    while d < n:
        pad = jnp.zeros(x.shape[:1] + (d,), x.dtype)
        x = x + jnp.concatenate([pad, x[:, :-d]], axis=1)
        d *= 2
    return x


def _route_kernel(col_ref, pos_ref, be_ref):
    c2 = col_ref[...]
    totals = [jnp.sum((c2 == c).astype(jnp.int32)) for c in range(X)]
    starts, ends_blk = [], []
    run = 0
    for c in range(X):
        pc = ((totals[c] + BM - 1) // BM) * BM
        starts.append(run)
        run = run + pc
        ends_blk.append(run // BM)
    pos = jnp.zeros((ROWS_2D, 128), jnp.int32)
    for c in range(X):
        m = (c2 == c).astype(jnp.int32)
        colcnt = jnp.sum(m, axis=0, keepdims=True)
        colexcl = _iscan_lanes(colcnt) - colcnt
        within = _iscan_rows(m)
        p = starts[c] + colexcl + within - 1
        pos = jnp.where(c2 == c, p, pos)
    pos_ref[...] = pos
    b = (lax.broadcasted_iota(jnp.int32, (X, 128), 0) * 128
         + lax.broadcasted_iota(jnp.int32, (X, 128), 1))
    e = sum([(b >= eb).astype(jnp.int32) for eb in ends_blk])
    be_ref[...] = jnp.minimum(e, X - 1)


def _route(colors2d):
    return pl.pallas_call(
        _route_kernel,
        out_shape=(
            jax.ShapeDtypeStruct((ROWS_2D, 128), jnp.int32),
            jax.ShapeDtypeStruct((X, 128), jnp.int32),
        ),
    )(colors2d)


def _sc_wid():
    return lax.axis_index("s") * 2 + lax.axis_index("c")


RPW = ROWS_2D // NW


def _worker_rows():
    wid = _sc_wid()
    r0 = wid * RPW
    nt = jnp.where(wid == NW - 1, ROWS_2D - (NW - 1) * RPW, RPW)
    return r0, nt


def _scatter_body(pos_hbm, ea_hbm, xs_hbm, idx_v, rows_v, sem_l, sem_s):
    r0, nt = _worker_rows()

    def loads(t, b):
        r = r0 + t
        pltpu.async_copy(pos_hbm.at[r], idx_v.at[b], sem_l)
        pltpu.async_copy(ea_hbm.at[pl.ds(r * 128, 128)], rows_v.at[b], sem_l)

    def wait_loads(t, b):
        r = r0 + t
        pltpu.make_async_copy(pos_hbm.at[r], idx_v.at[b], sem_l).wait()
        pltpu.make_async_copy(ea_hbm.at[pl.ds(r * 128, 128)], rows_v.at[b],
                              sem_l).wait()

    def wait_scatter(b):
        pltpu.make_async_copy(rows_v.at[b], xs_hbm.at[idx_v.at[b]],
                              sem_s).wait()

    loads(0, 0)
    loads(1, 1)

    def tile(t, carry):
        b = lax.rem(t, 4)

        @pl.when(t >= 2)
        def _():
            wait_scatter(lax.rem(t - 2, 4))

        @pl.when(t + 2 < nt)
        def _():
            loads(t + 2, lax.rem(t + 2, 4))

        wait_loads(t, b)
        pltpu.async_copy(rows_v.at[b], xs_hbm.at[idx_v.at[b]], sem_s)
        return carry

    lax.fori_loop(0, nt, tile, 0)
    wait_scatter(lax.rem(nt - 2, 4))
    wait_scatter(lax.rem(nt - 1, 4))


def _scatter(pos2d, x128):
    mesh = plsc.VectorSubcoreMesh(core_axis_name="c", subcore_axis_name="s")
    run = functools.partial(
        pl.kernel,
        mesh=mesh,
        out_type=jax.ShapeDtypeStruct((E_PAD, F), jnp.float32),
        scratch_types=[
            pltpu.VMEM((4, 128), jnp.int32),
            pltpu.VMEM((4, 128, F), jnp.float32),
            pltpu.SemaphoreType.DMA,
            pltpu.SemaphoreType.DMA,
        ],
    )(_scatter_body)
    return run(pos2d, x128)


def _mlp_kernel(be_ref, x_ref, w1_ref, b1_ref, w2_ref, b2_ref, y_ref):
    e = be_ref[pl.program_id(0)]
    x = x_ref[...][:, :G].astype(jnp.bfloat16)
    w1 = w1_ref[pl.ds(e, 1)][0]
    h = (jnp.dot(x, w1, preferred_element_type=jnp.float32)
         + b1_ref[pl.ds(e, 1)])
    sp = jnp.maximum(h, 0.0) + jnp.log(1.0 + jnp.exp(-jnp.abs(h))) - SHIFT
    w2 = w2_ref[pl.ds(e, 1)][0]
    y_ref[...] = (jnp.dot(sp.astype(jnp.bfloat16), w2,
                          preferred_element_type=jnp.float32)
                  + b2_ref[pl.ds(e, 1)])


def _mlp(block_expert, x_sorted, W1, b1, W2, b2):
    grid_spec = pltpu.PrefetchScalarGridSpec(
        num_scalar_prefetch=1,
        grid=(NB,),
        in_specs=[
            pl.BlockSpec((BM, F), lambda i, be: (i, 0)),
            pl.BlockSpec((X, G, F), lambda i, be: (0, 0, 0)),
            pl.BlockSpec((X, F), lambda i, be: (0, 0)),
            pl.BlockSpec((X, F, F), lambda i, be: (0, 0, 0)),
            pl.BlockSpec((X, F), lambda i, be: (0, 0)),
        ],
        out_specs=pl.BlockSpec((BM, F), lambda i, be: (i, 0)),
    )
    return pl.pallas_call(
        _mlp_kernel,
        grid_spec=grid_spec,
        out_shape=jax.ShapeDtypeStruct((E_PAD, F), jnp.float32),
        compiler_params=pltpu.CompilerParams(
            dimension_semantics=("parallel",)),
    )(block_expert, x_sorted, W1, b1, W2, b2)


def _gather_body(pos_hbm, ys_hbm, out_hbm, idx_v, rows_v, sem_i, sem_g, sem_s):
    r0, nt = _worker_rows()

    def load_idx(t, b):
        pltpu.async_copy(pos_hbm.at[r0 + t], idx_v.at[b], sem_i)

    def wait_idx(t, b):
        pltpu.make_async_copy(pos_hbm.at[r0 + t], idx_v.at[b], sem_i).wait()

    def gath(b):
        pltpu.async_copy(ys_hbm.at[idx_v.at[b]], rows_v.at[b], sem_g)

    def wait_gath(b):
        pltpu.make_async_copy(ys_hbm.at[idx_v.at[b]], rows_v.at[b],
                              sem_g).wait()

    def store(t, b):
        pltpu.async_copy(rows_v.at[b], out_hbm.at[pl.ds((r0 + t) * 128, 128)],
                         sem_s)

    def wait_store(t, b):
        pltpu.make_async_copy(rows_v.at[b],
                              out_hbm.at[pl.ds((r0 + t) * 128, 128)],
                              sem_s).wait()

    load_idx(0, 0)
    load_idx(1, 1)
    wait_idx(0, 0)
    gath(0)

    def tile(t, carry):
        b = lax.rem(t, 4)

        @pl.when(t >= 2)
        def _():
            wait_store(t - 2, lax.rem(t - 2, 4))

        @pl.when(t + 2 < nt)
        def _():
            load_idx(t + 2, lax.rem(t + 2, 4))

        @pl.when(t + 1 < nt)
        def _():
            b1 = lax.rem(t + 1, 4)
            wait_idx(t + 1, b1)
            gath(b1)

        wait_gath(b)
        store(t, b)
        return carry

    lax.fori_loop(0, nt, tile, 0)
    wait_store(nt - 2, lax.rem(nt - 2, 4))
    wait_store(nt - 1, lax.rem(nt - 1, 4))


def _gather(pos2d, y_sorted):
    mesh = plsc.VectorSubcoreMesh(core_axis_name="c", subcore_axis_name="s")
    run = functools.partial(
        pl.kernel,
        mesh=mesh,
        out_type=jax.ShapeDtypeStruct((E, F), jnp.float32),
        scratch_types=[
            pltpu.VMEM((4, 128), jnp.int32),
            pltpu.VMEM((4, 128, F), jnp.float32),
            pltpu.SemaphoreType.DMA,
            pltpu.SemaphoreType.DMA,
            pltpu.SemaphoreType.DMA,
        ],
    )(_gather_body)
    return run(pos2d, y_sorted)


def kernel(edge_attr, colors, W1, b1, W2, b2):
    pos2d, be2d = _route(colors.reshape(ROWS_2D, 128))
    block_expert = be2d.reshape(X * 128)
    x128 = jnp.concatenate(
        [edge_attr, jnp.zeros((E, F - G), jnp.float32)], axis=1)
    x_sorted = _scatter(pos2d, x128)
    y_sorted = _mlp(block_expert, x_sorted, W1.astype(jnp.bfloat16),
                    b1, W2.astype(jnp.bfloat16), b2)
    return _gather(pos2d, y_sorted)


# speedup vs baseline: 1.0010x; 1.0010x over previous
import functools

import jax
import jax.numpy as jnp
from jax import lax
from jax.experimental import pallas as pl
from jax.experimental.pallas import tpu as pltpu
from jax.experimental.pallas import tpu_sc as plsc

E = 160000
G = 50
F = 128
X = 8
BM = 2048
E_PAD = ((E + X * (BM - 1)) // BM) * BM
NB = E_PAD // BM
NW = 32
CHUNK = E // NW
NT = CHUNK // 128
TAIL = CHUNK - NT * 128
ROWS_2D = E // 128
SHIFT = 0.6931471805599453


def _iscan_rows(x):
    n = x.shape[0]
    d = 1
    while d < n:
        pad = jnp.zeros((d,) + x.shape[1:], x.dtype)
        x = x + jnp.concatenate([pad, x[:-d]], axis=0)
        d *= 2
    return x


def _iscan_lanes(x):
    n = x.shape[1]
    d = 1
    while d < n:
        pad = jnp.zeros(x.shape[:1] + (d,), x.dtype)
        x = x + jnp.concatenate([pad, x[:, :-d]], axis=1)
        d *= 2
    return x


def _route_kernel(col_ref, pos_ref, be_ref):
    c2 = col_ref[...]
    totals = [jnp.sum((c2 == c).astype(jnp.int32)) for c in range(X)]
    starts, ends_blk = [], []
    run = 0
    for c in range(X):
        pc = ((totals[c] + BM - 1) // BM) * BM
        starts.append(run)
        run = run + pc
        ends_blk.append(run // BM)
    pos = jnp.zeros((ROWS_2D, 128), jnp.int32)
    for c in range(X):
        m = (c2 == c).astype(jnp.int32)
        colcnt = jnp.sum(m, axis=0, keepdims=True)
        colexcl = _iscan_lanes(colcnt) - colcnt
        within = _iscan_rows(m)
        p = starts[c] + colexcl + within - 1
        pos = jnp.where(c2 == c, p, pos)
    pos_ref[...] = pos
    b = (lax.broadcasted_iota(jnp.int32, (X, 128), 0) * 128
         + lax.broadcasted_iota(jnp.int32, (X, 128), 1))
    e = sum([(b >= eb).astype(jnp.int32) for eb in ends_blk])
    be_ref[...] = jnp.minimum(e, X - 1)


def _route(colors2d):
    return pl.pallas_call(
        _route_kernel,
        out_shape=(
            jax.ShapeDtypeStruct((ROWS_2D, 128), jnp.int32),
            jax.ShapeDtypeStruct((X, 128), jnp.int32),
        ),
    )(colors2d)


def _sc_wid():
    return lax.axis_index("s") * 2 + lax.axis_index("c")


RPW = ROWS_2D // NW


def _worker_rows():
    wid = _sc_wid()
    r0 = wid * RPW
    nt = jnp.where(wid == NW - 1, ROWS_2D - (NW - 1) * RPW, RPW)
    return r0, nt


def _scatter_body(pos_hbm, ea_hbm, xs_hbm, idx_v, rows_v, sem_l, sem_s):
    r0, nt = _worker_rows()

    def loads(t, b):
        r = r0 + t
        pltpu.async_copy(pos_hbm.at[r], idx_v.at[b], sem_l)
        pltpu.async_copy(ea_hbm.at[pl.ds(r * 128, 128)], rows_v.at[b], sem_l)

    def wait_loads(t, b):
        r = r0 + t
        pltpu.make_async_copy(pos_hbm.at[r], idx_v.at[b], sem_l).wait()
        pltpu.make_async_copy(ea_hbm.at[pl.ds(r * 128, 128)], rows_v.at[b],
                              sem_l).wait()

    def wait_scatter(b):
        pltpu.make_async_copy(rows_v.at[b], xs_hbm.at[idx_v.at[b]],
                              sem_s).wait()

    loads(0, 0)
    loads(1, 1)

    def tile(t, carry):
        b = lax.rem(t, 4)

        @pl.when(t >= 2)
        def _():
            wait_scatter(lax.rem(t - 2, 4))

        @pl.when(t + 2 < nt)
        def _():
            loads(t + 2, lax.rem(t + 2, 4))

        wait_loads(t, b)
        pltpu.async_copy(rows_v.at[b], xs_hbm.at[idx_v.at[b]], sem_s)
        return carry

    lax.fori_loop(0, nt, tile, 0)
    wait_scatter(lax.rem(nt - 2, 4))
    wait_scatter(lax.rem(nt - 1, 4))


def _scatter(pos2d, x128):
    mesh = plsc.VectorSubcoreMesh(core_axis_name="c", subcore_axis_name="s")
    run = functools.partial(
        pl.kernel,
        mesh=mesh,
        out_type=jax.ShapeDtypeStruct((E_PAD, F), jnp.float32),
        scratch_types=[
            pltpu.VMEM((4, 128), jnp.int32),
            pltpu.VMEM((4, 128, F), jnp.float32),
            pltpu.SemaphoreType.DMA,
            pltpu.SemaphoreType.DMA,
        ],
    )(_scatter_body)
    return run(pos2d, x128)


def _mlp_kernel(be_ref, x_ref, w1_ref, b1_ref, w2_ref, b2_ref, y_ref):
    e = be_ref[pl.program_id(0)]
    x = x_ref[...][:, :G].astype(jnp.bfloat16)
    w1 = w1_ref[pl.ds(e, 1)][0]
    h = (jnp.dot(x, w1, preferred_element_type=jnp.float32)
         + b1_ref[pl.ds(e, 1)])
    sp = jnp.maximum(h, 0.0) + jnp.log(1.0 + jnp.exp(-jnp.abs(h))) - SHIFT
    w2 = w2_ref[pl.ds(e, 1)][0]
    y_ref[...] = (jnp.dot(sp.astype(jnp.bfloat16), w2,
                          preferred_element_type=jnp.float32)
                  + b2_ref[pl.ds(e, 1)])


def _mlp(block_expert, x_sorted, W1, b1, W2, b2):
    grid_spec = pltpu.PrefetchScalarGridSpec(
        num_scalar_prefetch=1,
        grid=(NB,),
        in_specs=[
            pl.BlockSpec((BM, F), lambda i, be: (i, 0)),
            pl.BlockSpec((X, G, F), lambda i, be: (0, 0, 0)),
            pl.BlockSpec((X, F), lambda i, be: (0, 0)),
            pl.BlockSpec((X, F, F), lambda i, be: (0, 0, 0)),
            pl.BlockSpec((X, F), lambda i, be: (0, 0)),
        ],
        out_specs=pl.BlockSpec((BM, F), lambda i, be: (i, 0)),
    )
    return pl.pallas_call(
        _mlp_kernel,
        grid_spec=grid_spec,
        out_shape=jax.ShapeDtypeStruct((E_PAD, F), jnp.float32),
        compiler_params=pltpu.CompilerParams(
            dimension_semantics=("parallel",)),
    )(block_expert, x_sorted, W1, b1, W2, b2)


def _gather_body(pos_hbm, ys_hbm, out_hbm, idx_v, rows_v, sem_i, sem_g, sem_s):
    r0, nt = _worker_rows()

    def load_idx(t, b):
        pltpu.async_copy(pos_hbm.at[r0 + t], idx_v.at[b], sem_i)

    def wait_idx(t, b):
        pltpu.make_async_copy(pos_hbm.at[r0 + t], idx_v.at[b], sem_i).wait()

    def gath(b):
        pltpu.async_copy(ys_hbm.at[idx_v.at[b]], rows_v.at[b], sem_g)

    def wait_gath(b):
        pltpu.make_async_copy(ys_hbm.at[idx_v.at[b]], rows_v.at[b],
                              sem_g).wait()

    def store(t, b):
        pltpu.async_copy(rows_v.at[b], out_hbm.at[pl.ds((r0 + t) * 128, 128)],
                         sem_s)

    def wait_store(t, b):
        pltpu.make_async_copy(rows_v.at[b],
                              out_hbm.at[pl.ds((r0 + t) * 128, 128)],
                              sem_s).wait()

    load_idx(0, 0)
    load_idx(1, 1)
    wait_idx(0, 0)
    gath(0)

    def tile(t, carry):
        b = lax.rem(t, 4)

        @pl.when(t >= 2)
        def _():
            wait_store(t - 2, lax.rem(t - 2, 4))

        @pl.when(t + 2 < nt)
        def _():
            load_idx(t + 2, lax.rem(t + 2, 4))

        @pl.when(t + 1 < nt)
        def _():
            b1 = lax.rem(t + 1, 4)
            wait_idx(t + 1, b1)
            gath(b1)

        wait_gath(b)
        store(t, b)
        return carry

    lax.fori_loop(0, nt, tile, 0)
    wait_store(nt - 2, lax.rem(nt - 2, 4))
    wait_store(nt - 1, lax.rem(nt - 1, 4))


def _gather(pos2d, y_sorted):
    mesh = plsc.VectorSubcoreMesh(core_axis_name="c", subcore_axis_name="s")
    run = functools.partial(
        pl.kernel,
        mesh=mesh,
        out_type=jax.ShapeDtypeStruct((E, F), jnp.float32),
        scratch_types=[
            pltpu.VMEM((4, 128), jnp.int32),
            pltpu.VMEM((4, 128, F), jnp.float32),
            pltpu.SemaphoreType.DMA,
            pltpu.SemaphoreType.DMA,
            pltpu.SemaphoreType.DMA,
        ],
    )(_gather_body)
    return run(pos2d, y_sorted)


def kernel(edge_attr, colors, W1, b1, W2, b2):
    pos2d, be2d = _route(colors.reshape(ROWS_2D, 128))
    block_expert = be2d.reshape(X * 128)
    x128 = jnp.pad(edge_attr, ((0, 0), (0, F - G)))
    x_sorted = _scatter(pos2d, x128)
    y_sorted = _mlp(block_expert, x_sorted, W1.astype(jnp.bfloat16),
                    b1, W2.astype(jnp.bfloat16), b2)
    return _gather(pos2d, y_sorted)
